# baseline (speedup 1.0000x reference)
import functools

import jax
import jax.numpy as jnp
from jax import lax
from jax.experimental import pallas as pl
from jax.experimental.pallas import tpu as pltpu
from jax.experimental.pallas import tpu_sc as plsc

N = 10000
D_IN = 128
D_HID = 128
D_OUT = 40
NUM_TARGET = 1000

NC = 2
NS = 16
CHUNK = 128


def _ceil_div(a, b):
    return -(-a // b)


def _mm_body(x_ref, w_ref, o_ref):
    o_ref[...] = jnp.dot(x_ref[...], w_ref[...],
                         preferred_element_type=jnp.float32)


def _tc_matmul(x, w, bm):
    m, k = x.shape
    n = w.shape[1]
    grid = m // bm
    return pl.pallas_call(
        _mm_body,
        grid=(grid,),
        in_specs=[
            pl.BlockSpec((bm, k), lambda i: (i, 0)),
            pl.BlockSpec((k, n), lambda i: (0, 0)),
        ],
        out_specs=pl.BlockSpec((bm, n), lambda i: (i, 0)),
        out_shape=jax.ShapeDtypeStruct((m, n), jnp.float32),
    )(x, w)


def _relu_mm_body(p0_ref, p1_ref, c_ref, w_ref, o_ref):
    a = jnp.maximum(p0_ref[...] + p1_ref[...] + c_ref[...], 0.0)
    o_ref[...] = jnp.dot(a, w_ref[...], preferred_element_type=jnp.float32)


def _tc_relu_matmul(parts, cvec, w, bm):
    d = parts.shape[1]
    n = w.shape[1]
    grid = N // bm
    nb = N // bm
    return pl.pallas_call(
        _relu_mm_body,
        grid=(grid,),
        in_specs=[
            pl.BlockSpec((bm, d), lambda i: (i, 0)),
            pl.BlockSpec((bm, d), lambda i, nb=nb: (i + nb, 0)),
            pl.BlockSpec((1, d), lambda i: (0, 0)),
            pl.BlockSpec((d, n), lambda i: (0, 0)),
        ],
        out_specs=pl.BlockSpec((bm, n), lambda i: (i, 0)),
        out_shape=jax.ShapeDtypeStruct((N, n), jnp.float32),
    )(parts, parts, cvec, w)


def _lsm_body(q_ref, b_ref, o_ref, rows_a, rows_b):
    z = (q_ref[0:NUM_TARGET, 0:D_OUT] + q_ref[rows_a:rows_b, 0:D_OUT]
         + b_ref[...])
    m = jnp.max(z, axis=-1, keepdims=True)
    e = jnp.exp(z - m)
    lse = jnp.log(jnp.sum(e, axis=-1, keepdims=True))
    o_ref[...] = z - m - lse


def _tc_logsoftmax(q, b2, rows_per_core):
    body = functools.partial(_lsm_body, rows_a=rows_per_core,
                             rows_b=rows_per_core + NUM_TARGET)
    return pl.pallas_call(
        body,
        grid=(1,),
        in_specs=[
            pl.BlockSpec(q.shape, lambda i: (0, 0)),
            pl.BlockSpec((1, D_OUT), lambda i: (0, 0)),
        ],
        out_specs=pl.BlockSpec((NUM_TARGET, D_OUT), lambda i: (0, 0)),
        out_shape=jax.ShapeDtypeStruct((NUM_TARGET, D_OUT), jnp.float32),
    )(q, b2.reshape(1, D_OUT))


def _sc_agg_body(n_rows, acc_rows, d, chunks_per_tile, out_tiles,
                 h_hbm, src_hbm, dst_hbm, zeros_hbm, out_hbm,
                 idx_s, idx_d, rows0, rows1,
                 acc, gsem0, gsem1, ssem0, ssem1):
    c = lax.axis_index("c")
    s = lax.axis_index("s")
    ch = chunks_per_tile
    half = ch // 2
    zrows = acc_rows // NS
    orows = n_rows // out_tiles
    wid = c * NS + s
    pltpu.sync_copy(zeros_hbm, acc.at[pl.ds(s * zrows, zrows)])
    plsc.subcore_barrier()

    rows = (rows0, rows1)
    gsems = (gsem0, gsem1)
    ssems = (ssem0, ssem1)

    def fire_gather(k, b):
        pltpu.async_copy(h_hbm.at[idx_s.at[k]], rows[b], gsems[b])

    def wait_gather(k, b):
        pltpu.make_async_copy(h_hbm.at[idx_s.at[k]], rows[b],
                              gsems[b]).wait()

    def fire_scatter(k, b):
        pltpu.async_copy(rows[b], acc.at[idx_d.at[k]], ssems[b],
                         add=True)

    def wait_scatter(k, b):
        pltpu.make_async_copy(rows[b], acc.at[idx_d.at[k]],
                              ssems[b]).wait()

    for seg in range(2):
        seg_base = seg * half
        pltpu.sync_copy(src_hbm.at[pl.ds(wid * ch + seg_base, half)],
                        idx_s)
        pltpu.sync_copy(dst_hbm.at[pl.ds(wid * ch + seg_base, half)],
                        idx_d)
        fire_gather(0, 0)

        @pl.loop(0, half, step=2)
        def _(k):
            for b in range(2):
                kk = k + b
                wait_gather(kk, b)
                fire_scatter(kk, b)

                @pl.when(kk >= 1)
                def _():
                    wait_scatter(kk - 1, 1 - b)

                @pl.when(kk + 1 < half)
                def _():
                    fire_gather(kk + 1, 1 - b)

        wait_scatter(half - 1, (half - 1) % 2)

    plsc.subcore_barrier()

    @pl.when(s < out_tiles)
    def _():
        pltpu.sync_copy(acc.at[pl.ds(s * orows, orows)],
                        out_hbm.at[pl.ds((c * n_rows) + s * orows, orows)])


def _sc_aggregate(h, src, dst, n_rows, acc_rows, chunks_per_tile, out_tiles):
    d = h.shape[1]
    zrows = acc_rows // NS
    zeros = jnp.zeros((zrows, d), jnp.float32)
    src2d = src.reshape(-1, CHUNK)
    dst2d = dst.reshape(-1, CHUNK)
    mesh = plsc.VectorSubcoreMesh(core_axis_name="c", subcore_axis_name="s")
    body = functools.partial(_sc_agg_body, n_rows, acc_rows, d,
                             chunks_per_tile, out_tiles)
    return pl.kernel(
        body,
        out_type=jax.ShapeDtypeStruct((NC * n_rows, d), jnp.float32),
        mesh=mesh,
        scratch_types=[
            pltpu.VMEM((chunks_per_tile // 2, CHUNK), jnp.int32),
            pltpu.VMEM((chunks_per_tile // 2, CHUNK), jnp.int32),
            pltpu.VMEM((CHUNK, d), jnp.float32),
            pltpu.VMEM((CHUNK, d), jnp.float32),
            pltpu.VMEM_SHARED((acc_rows, d), jnp.float32),
            pltpu.SemaphoreType.DMA,
            pltpu.SemaphoreType.DMA,
            pltpu.SemaphoreType.DMA,
            pltpu.SemaphoreType.DMA,
        ],
    )(h, src2d, dst2d, zeros)


def _pad_edges(src, dst, e, e_pad, junk_lo, junk_n):
    pad = e_pad - e
    junk = junk_lo + (jnp.arange(pad, dtype=jnp.int32) % junk_n)
    srcp = jnp.concatenate([src, jnp.zeros((pad,), jnp.int32)])
    dstp = jnp.concatenate([dst, junk])
    return srcp, dstp


def kernel(x, edge_index_1, edge_index_2, num_target,
           W1, b1, gamma1, beta1, W2, b2):
    eps = 1e-5
    scale = gamma1 / jnp.sqrt(1.0 + eps)
    w1s = W1 * scale[None, :]
    cvec = (b1 * scale + beta1).reshape(1, D_HID)

    e1 = edge_index_1.shape[1]
    e2 = edge_index_2.shape[1]
    ch1 = _ceil_div(_ceil_div(e1, NC * NS * CHUNK), 16) * 16
    ch2 = _ceil_div(_ceil_div(e2, NC * NS * CHUNK), 16) * 16
    e1p = NC * NS * ch1 * CHUNK
    e2p = NC * NS * ch2 * CHUNK

    acc1_rows = _ceil_div(N + 1, NS * 8) * NS * 8
    src1, dst1 = _pad_edges(edge_index_1[0], edge_index_1[1], e1, e1p,
                            N, acc1_rows - N)

    acc2_rows = 2048
    d2 = edge_index_2[1]
    dst2 = jnp.where(d2 < NUM_TARGET, d2,
                     NUM_TARGET + lax.rem(d2, acc2_rows - NUM_TARGET))
    src2, dst2 = _pad_edges(edge_index_2[0], dst2, e2, e2p,
                            NUM_TARGET, acc2_rows - NUM_TARGET)

    w2p = jnp.pad(W2, ((0, 0), (0, 128 - D_OUT)))

    h = _tc_matmul(x, w1s, bm=1000)
    parts1 = _sc_aggregate(h, src1, dst1, N, acc1_rows, ch1, 10)
    h2 = _tc_relu_matmul(parts1, cvec, w2p, bm=1000)
    parts2 = _sc_aggregate(h2, src2, dst2, acc2_rows, acc2_rows, ch2,
                           NS)
    return _tc_logsoftmax(parts2, b2, acc2_rows)


# speedup vs baseline: 1.0128x; 1.0128x over previous
import functools

import jax
import jax.numpy as jnp
from jax import lax
from jax.experimental import pallas as pl
from jax.experimental.pallas import tpu as pltpu
from jax.experimental.pallas import tpu_sc as plsc

N = 10000
D_IN = 128
D_HID = 128
D_OUT = 40
NUM_TARGET = 1000

NC = 2
NS = 16
CHUNK = 128


def _ceil_div(a, b):
    return -(-a // b)


def _mm_body(x_ref, w_ref, o_ref):
    o_ref[...] = jnp.dot(x_ref[...], w_ref[...],
                         preferred_element_type=jnp.float32)


def _tc_matmul(x, w, bm):
    m, k = x.shape
    n = w.shape[1]
    grid = m // bm
    return pl.pallas_call(
        _mm_body,
        grid=(grid,),
        in_specs=[
            pl.BlockSpec((bm, k), lambda i: (i, 0)),
            pl.BlockSpec((k, n), lambda i: (0, 0)),
        ],
        out_specs=pl.BlockSpec((bm, n), lambda i: (i, 0)),
        out_shape=jax.ShapeDtypeStruct((m, n), jnp.float32),
    )(x, w)


def _relu_mm_body(p0_ref, p1_ref, c_ref, w_ref, o_ref):
    a = jnp.maximum(p0_ref[...] + p1_ref[...] + c_ref[...], 0.0)
    o_ref[...] = jnp.dot(a, w_ref[...], preferred_element_type=jnp.float32)


def _tc_relu_matmul(parts, cvec, w, bm):
    d = parts.shape[1]
    n = w.shape[1]
    grid = N // bm
    nb = N // bm
    return pl.pallas_call(
        _relu_mm_body,
        grid=(grid,),
        in_specs=[
            pl.BlockSpec((bm, d), lambda i: (i, 0)),
            pl.BlockSpec((bm, d), lambda i, nb=nb: (i + nb, 0)),
            pl.BlockSpec((1, d), lambda i: (0, 0)),
            pl.BlockSpec((d, n), lambda i: (0, 0)),
        ],
        out_specs=pl.BlockSpec((bm, n), lambda i: (i, 0)),
        out_shape=jax.ShapeDtypeStruct((N, n), jnp.float32),
    )(parts, parts, cvec, w)


def _lsm_body(q_ref, b_ref, o_ref, rows_a, rows_b):
    z = (q_ref[0:NUM_TARGET, 0:D_OUT] + q_ref[rows_a:rows_b, 0:D_OUT]
         + b_ref[...])
    m = jnp.max(z, axis=-1, keepdims=True)
    e = jnp.exp(z - m)
    lse = jnp.log(jnp.sum(e, axis=-1, keepdims=True))
    o_ref[...] = z - m - lse


def _tc_logsoftmax(q, b2, rows_per_core):
    body = functools.partial(_lsm_body, rows_a=rows_per_core,
                             rows_b=rows_per_core + NUM_TARGET)
    return pl.pallas_call(
        body,
        grid=(1,),
        in_specs=[
            pl.BlockSpec(q.shape, lambda i: (0, 0)),
            pl.BlockSpec((1, D_OUT), lambda i: (0, 0)),
        ],
        out_specs=pl.BlockSpec((NUM_TARGET, D_OUT), lambda i: (0, 0)),
        out_shape=jax.ShapeDtypeStruct((NUM_TARGET, D_OUT), jnp.float32),
    )(q, b2.reshape(1, D_OUT))


def _sc_agg_body(n_rows, acc_rows, d, chunks_per_tile, out_tiles,
                 h_hbm, src_hbm, dst_hbm, zeros_hbm, out_hbm,
                 idx_s, idx_d, rows0, rows1,
                 acc, gsem0, gsem1, ssem0, ssem1):
    c = lax.axis_index("c")
    s = lax.axis_index("s")
    ch = chunks_per_tile
    half = ch // 2
    zrows = acc_rows // NS
    orows = n_rows // out_tiles
    wid = c * NS + s
    pltpu.sync_copy(zeros_hbm, acc.at[pl.ds(s * zrows, zrows)])
    plsc.subcore_barrier()

    rows = (rows0, rows1)
    gsems = (gsem0, gsem1)
    ssems = (ssem0, ssem1)

    def fire_gather(k, b):
        pltpu.async_copy(h_hbm.at[idx_s.at[k]], rows[b], gsems[b])

    def wait_gather(k, b):
        pltpu.make_async_copy(h_hbm.at[idx_s.at[k]], rows[b],
                              gsems[b]).wait()

    def fire_scatter(k, b):
        pltpu.async_copy(rows[b], acc.at[idx_d.at[k]], ssems[b],
                         add=True)

    def wait_scatter(k, b):
        pltpu.make_async_copy(rows[b], acc.at[idx_d.at[k]],
                              ssems[b]).wait()

    for seg in range(2):
        seg_base = seg * half
        pltpu.sync_copy(src_hbm.at[pl.ds(wid * ch + seg_base, half)],
                        idx_s)
        pltpu.sync_copy(dst_hbm.at[pl.ds(wid * ch + seg_base, half)],
                        idx_d)
        fire_gather(0, 0)

        @pl.loop(0, half, step=2)
        def _(k):
            for b in range(2):
                kk = k + b

                @pl.when(kk + 1 < half)
                def _():
                    fire_gather(kk + 1, 1 - b)

                wait_gather(kk, b)
                pltpu.sync_copy(rows[b], acc.at[idx_d.at[kk]], add=True)

    plsc.subcore_barrier()

    @pl.when(s < out_tiles)
    def _():
        pltpu.sync_copy(acc.at[pl.ds(s * orows, orows)],
                        out_hbm.at[pl.ds((c * n_rows) + s * orows, orows)])


def _sc_aggregate(h, src, dst, n_rows, acc_rows, chunks_per_tile, out_tiles):
    d = h.shape[1]
    zrows = acc_rows // NS
    zeros = jnp.zeros((zrows, d), jnp.float32)
    src2d = src.reshape(-1, CHUNK)
    dst2d = dst.reshape(-1, CHUNK)
    mesh = plsc.VectorSubcoreMesh(core_axis_name="c", subcore_axis_name="s")
    body = functools.partial(_sc_agg_body, n_rows, acc_rows, d,
                             chunks_per_tile, out_tiles)
    return pl.kernel(
        body,
        out_type=jax.ShapeDtypeStruct((NC * n_rows, d), jnp.float32),
        mesh=mesh,
        scratch_types=[
            pltpu.VMEM((chunks_per_tile // 2, CHUNK), jnp.int32),
            pltpu.VMEM((chunks_per_tile // 2, CHUNK), jnp.int32),
            pltpu.VMEM((CHUNK, d), jnp.float32),
            pltpu.VMEM((CHUNK, d), jnp.float32),
            pltpu.VMEM_SHARED((acc_rows, d), jnp.float32),
            pltpu.SemaphoreType.DMA,
            pltpu.SemaphoreType.DMA,
            pltpu.SemaphoreType.DMA,
            pltpu.SemaphoreType.DMA,
        ],
    )(h, src2d, dst2d, zeros)


def _pad_edges(src, dst, e, e_pad, junk_lo, junk_n):
    pad = e_pad - e
    junk = junk_lo + (jnp.arange(pad, dtype=jnp.int32) % junk_n)
    srcp = jnp.concatenate([src, jnp.zeros((pad,), jnp.int32)])
    dstp = jnp.concatenate([dst, junk])
    return srcp, dstp


def kernel(x, edge_index_1, edge_index_2, num_target,
           W1, b1, gamma1, beta1, W2, b2):
    eps = 1e-5
    scale = gamma1 / jnp.sqrt(1.0 + eps)
    w1s = W1 * scale[None, :]
    cvec = (b1 * scale + beta1).reshape(1, D_HID)

    e1 = edge_index_1.shape[1]
    e2 = edge_index_2.shape[1]
    ch1 = _ceil_div(_ceil_div(e1, NC * NS * CHUNK), 16) * 16
    ch2 = _ceil_div(_ceil_div(e2, NC * NS * CHUNK), 16) * 16
    e1p = NC * NS * ch1 * CHUNK
    e2p = NC * NS * ch2 * CHUNK

    acc1_rows = _ceil_div(N + 1, NS * 8) * NS * 8
    src1, dst1 = _pad_edges(edge_index_1[0], edge_index_1[1], e1, e1p,
                            N, acc1_rows - N)

    acc2_rows = 2048
    d2 = edge_index_2[1]
    dst2 = jnp.where(d2 < NUM_TARGET, d2,
                     NUM_TARGET + lax.rem(d2, acc2_rows - NUM_TARGET))
    src2, dst2 = _pad_edges(edge_index_2[0], dst2, e2, e2p,
                            NUM_TARGET, acc2_rows - NUM_TARGET)

    w2p = jnp.pad(W2, ((0, 0), (0, 128 - D_OUT)))

    h = _tc_matmul(x, w1s, bm=1000)
    parts1 = _sc_aggregate(h, src1, dst1, N, acc1_rows, ch1, 10)
    h2 = _tc_relu_matmul(parts1, cvec, w2p, bm=1000)
    parts2 = _sc_aggregate(h2, src2, dst2, acc2_rows, acc2_rows, ch2,
                           NS)
    return _tc_logsoftmax(parts2, b2, acc2_rows)


# speedup vs baseline: 3.0483x; 3.0098x over previous
import functools

import jax
import jax.numpy as jnp
from jax import lax
from jax.experimental import pallas as pl
from jax.experimental.pallas import tpu as pltpu
from jax.experimental.pallas import tpu_sc as plsc

N = 10000
D_IN = 128
D_HID = 128
D_OUT = 40
NUM_TARGET = 1000

NC = 2
NS = 16
CHUNK = 128


def _ceil_div(a, b):
    return -(-a // b)


def _mm_body(x_ref, w_ref, o_ref):
    o_ref[...] = jnp.dot(x_ref[...], w_ref[...],
                         preferred_element_type=jnp.float32)


def _tc_matmul(x, w, bm):
    m, k = x.shape
    n = w.shape[1]
    grid = m // bm
    return pl.pallas_call(
        _mm_body,
        grid=(grid,),
        in_specs=[
            pl.BlockSpec((bm, k), lambda i: (i, 0)),
            pl.BlockSpec((k, n), lambda i: (0, 0)),
        ],
        out_specs=pl.BlockSpec((bm, n), lambda i: (i, 0)),
        out_shape=jax.ShapeDtypeStruct((m, n), jnp.float32),
    )(x, w)


def _relu_mm_body(p0_ref, p1_ref, c_ref, w_ref, o_ref):
    a = jnp.maximum(p0_ref[...] + p1_ref[...] + c_ref[...], 0.0)
    o_ref[...] = jnp.dot(a, w_ref[...], preferred_element_type=jnp.float32)


def _tc_relu_matmul(parts, cvec, w, bm):
    d = parts.shape[1]
    n = w.shape[1]
    grid = N // bm
    nb = N // bm
    return pl.pallas_call(
        _relu_mm_body,
        grid=(grid,),
        in_specs=[
            pl.BlockSpec((bm, d), lambda i: (i, 0)),
            pl.BlockSpec((bm, d), lambda i, nb=nb: (i + nb, 0)),
            pl.BlockSpec((1, d), lambda i: (0, 0)),
            pl.BlockSpec((d, n), lambda i: (0, 0)),
        ],
        out_specs=pl.BlockSpec((bm, n), lambda i: (i, 0)),
        out_shape=jax.ShapeDtypeStruct((N, n), jnp.float32),
    )(parts, parts, cvec, w)


def _lsm_body(q_ref, b_ref, o_ref, rows_a, rows_b):
    z = (q_ref[0:NUM_TARGET, 0:D_OUT] + q_ref[rows_a:rows_b, 0:D_OUT]
         + b_ref[...])
    m = jnp.max(z, axis=-1, keepdims=True)
    e = jnp.exp(z - m)
    lse = jnp.log(jnp.sum(e, axis=-1, keepdims=True))
    o_ref[...] = z - m - lse


def _tc_logsoftmax(q, b2, rows_per_core):
    body = functools.partial(_lsm_body, rows_a=rows_per_core,
                             rows_b=rows_per_core + NUM_TARGET)
    return pl.pallas_call(
        body,
        grid=(1,),
        in_specs=[
            pl.BlockSpec(q.shape, lambda i: (0, 0)),
            pl.BlockSpec((1, D_OUT), lambda i: (0, 0)),
        ],
        out_specs=pl.BlockSpec((NUM_TARGET, D_OUT), lambda i: (0, 0)),
        out_shape=jax.ShapeDtypeStruct((NUM_TARGET, D_OUT), jnp.float32),
    )(q, b2.reshape(1, D_OUT))


def _sc_agg_body(n_rows, acc_rows, d, chunks_per_tile, out_tiles,
                 h_hbm, src_hbm, dst_hbm, zeros_hbm, out_hbm,
                 idx_s0, idx_s1, idx_d0, idx_d1, rows0, rows1,
                 acc, isem0, isem1, gsem0, gsem1):
    c = lax.axis_index("c")
    s = lax.axis_index("s")
    ch = chunks_per_tile
    zrows = acc_rows // NS
    orows = n_rows // out_tiles
    pltpu.sync_copy(zeros_hbm, acc.at[pl.ds(s * zrows, zrows)])
    plsc.subcore_barrier()

    per_tile = ch * CHUNK
    base = (c * NS + s) * per_tile
    idx_s = (idx_s0, idx_s1)
    idx_d = (idx_d0, idx_d1)
    rows = (rows0, rows1)
    isems = (isem0, isem1)
    gsems = (gsem0, gsem1)

    def fire_idx(k, b):
        off = base + k * CHUNK
        pltpu.async_copy(src_hbm.at[pl.ds(off, CHUNK)], idx_s[b], isems[b])
        pltpu.async_copy(dst_hbm.at[pl.ds(off, CHUNK)], idx_d[b], isems[b])

    def wait_idx(k, b):
        off = base + k * CHUNK
        pltpu.make_async_copy(src_hbm.at[pl.ds(off, CHUNK)], idx_s[b],
                              isems[b]).wait()
        pltpu.make_async_copy(dst_hbm.at[pl.ds(off, CHUNK)], idx_d[b],
                              isems[b]).wait()

    fire_idx(0, 0)
    wait_idx(0, 0)
    pltpu.async_copy(h_hbm.at[idx_s[0]], rows[0], gsems[0])
    fire_idx(1, 1)

    @pl.loop(0, ch, step=2)
    def _(k):
        for b in range(2):
            kk = k + b
            nb = 1 - b

            @pl.when(kk + 1 < ch)
            def _():
                wait_idx(kk + 1, nb)
                pltpu.async_copy(h_hbm.at[idx_s[nb]], rows[nb], gsems[nb])

            pltpu.make_async_copy(h_hbm.at[idx_s[b]], rows[b],
                                  gsems[b]).wait()
            pltpu.sync_copy(rows[b], acc.at[idx_d[b]], add=True)

            @pl.when(kk + 2 < ch)
            def _():
                fire_idx(kk + 2, b)

    plsc.subcore_barrier()

    @pl.when(s < out_tiles)
    def _():
        pltpu.sync_copy(acc.at[pl.ds(s * orows, orows)],
                        out_hbm.at[pl.ds((c * n_rows) + s * orows, orows)])


def _sc_aggregate(h, src, dst, n_rows, acc_rows, chunks_per_tile, out_tiles):
    d = h.shape[1]
    zrows = acc_rows // NS
    zeros = jnp.zeros((zrows, d), jnp.float32)
    mesh = plsc.VectorSubcoreMesh(core_axis_name="c", subcore_axis_name="s")
    body = functools.partial(_sc_agg_body, n_rows, acc_rows, d,
                             chunks_per_tile, out_tiles)
    return pl.kernel(
        body,
        out_type=jax.ShapeDtypeStruct((NC * n_rows, d), jnp.float32),
        mesh=mesh,
        scratch_types=[
            pltpu.VMEM((CHUNK,), jnp.int32),
            pltpu.VMEM((CHUNK,), jnp.int32),
            pltpu.VMEM((CHUNK,), jnp.int32),
            pltpu.VMEM((CHUNK,), jnp.int32),
            pltpu.VMEM((CHUNK, d), jnp.float32),
            pltpu.VMEM((CHUNK, d), jnp.float32),
            pltpu.VMEM_SHARED((acc_rows, d), jnp.float32),
            pltpu.SemaphoreType.DMA,
            pltpu.SemaphoreType.DMA,
            pltpu.SemaphoreType.DMA,
            pltpu.SemaphoreType.DMA,
        ],
    )(h, src, dst, zeros)


def _pad_edges(src, dst, e, e_pad, junk_lo, junk_n):
    pad = e_pad - e
    junk = junk_lo + (jnp.arange(pad, dtype=jnp.int32) % junk_n)
    srcp = jnp.concatenate([src, jnp.zeros((pad,), jnp.int32)])
    dstp = jnp.concatenate([dst, junk])
    return srcp, dstp


def kernel(x, edge_index_1, edge_index_2, num_target,
           W1, b1, gamma1, beta1, W2, b2):
    eps = 1e-5
    scale = gamma1 / jnp.sqrt(1.0 + eps)
    w1s = W1 * scale[None, :]
    cvec = (b1 * scale + beta1).reshape(1, D_HID)

    e1 = edge_index_1.shape[1]
    e2 = edge_index_2.shape[1]
    ch1 = _ceil_div(_ceil_div(e1, NC * NS * CHUNK), 2) * 2
    ch2 = _ceil_div(_ceil_div(e2, NC * NS * CHUNK), 2) * 2
    e1p = NC * NS * ch1 * CHUNK
    e2p = NC * NS * ch2 * CHUNK

    acc1_rows = _ceil_div(N + 1, NS * 8) * NS * 8
    src1, dst1 = _pad_edges(edge_index_1[0], edge_index_1[1], e1, e1p,
                            N, acc1_rows - N)

    acc2_rows = 2048
    d2 = edge_index_2[1]
    dst2 = jnp.where(d2 < NUM_TARGET, d2,
                     NUM_TARGET + lax.rem(d2, acc2_rows - NUM_TARGET))
    src2, dst2 = _pad_edges(edge_index_2[0], dst2, e2, e2p,
                            NUM_TARGET, acc2_rows - NUM_TARGET)

    w2p = jnp.pad(W2, ((0, 0), (0, 128 - D_OUT)))

    h = _tc_matmul(x, w1s, bm=1000)
    parts1 = _sc_aggregate(h, src1, dst1, N, acc1_rows, ch1, 10)
    h2 = _tc_relu_matmul(parts1, cvec, w2p, bm=1000)
    parts2 = _sc_aggregate(h2, src2, dst2, acc2_rows, acc2_rows, ch2,
                           NS)
    return _tc_logsoftmax(parts2, b2, acc2_rows)


# speedup vs baseline: 4.1185x; 1.3511x over previous
import functools

import jax
import jax.numpy as jnp
from jax import lax
from jax.experimental import pallas as pl
from jax.experimental.pallas import tpu as pltpu
from jax.experimental.pallas import tpu_sc as plsc

N = 10000
D_IN = 128
D_HID = 128
D_OUT = 40
NUM_TARGET = 1000

NC = 2
NS = 16
CHUNK = 128


def _ceil_div(a, b):
    return -(-a // b)


def _mm_body(x_ref, w_ref, o_ref):
    o_ref[...] = jnp.dot(x_ref[...], w_ref[...],
                         preferred_element_type=jnp.float32)


def _tc_matmul(x, w, bm):
    m, k = x.shape
    n = w.shape[1]
    grid = m // bm
    return pl.pallas_call(
        _mm_body,
        grid=(grid,),
        in_specs=[
            pl.BlockSpec((bm, k), lambda i: (i, 0)),
            pl.BlockSpec((k, n), lambda i: (0, 0)),
        ],
        out_specs=pl.BlockSpec((bm, n), lambda i: (i, 0)),
        out_shape=jax.ShapeDtypeStruct((m, n), jnp.float32),
    )(x, w)


def _relu_mm_body(p0_ref, p1_ref, c_ref, w_ref, o_ref):
    a = jnp.maximum(p0_ref[...] + p1_ref[...] + c_ref[...], 0.0)
    o_ref[...] = jnp.dot(a, w_ref[...], preferred_element_type=jnp.float32)


def _tc_relu_matmul(parts, cvec, w, bm):
    d = parts.shape[1]
    n = w.shape[1]
    grid = N // bm
    nb = N // bm
    return pl.pallas_call(
        _relu_mm_body,
        grid=(grid,),
        in_specs=[
            pl.BlockSpec((bm, d), lambda i: (i, 0)),
            pl.BlockSpec((bm, d), lambda i, nb=nb: (i + nb, 0)),
            pl.BlockSpec((1, d), lambda i: (0, 0)),
            pl.BlockSpec((d, n), lambda i: (0, 0)),
        ],
        out_specs=pl.BlockSpec((bm, n), lambda i: (i, 0)),
        out_shape=jax.ShapeDtypeStruct((N, n), jnp.float32),
    )(parts, parts, cvec, w)


def _lsm_body(q_ref, b_ref, o_ref, rows_a, rows_b):
    z = (q_ref[0:NUM_TARGET, 0:D_OUT] + q_ref[rows_a:rows_b, 0:D_OUT]
         + b_ref[...])
    m = jnp.max(z, axis=-1, keepdims=True)
    e = jnp.exp(z - m)
    lse = jnp.log(jnp.sum(e, axis=-1, keepdims=True))
    o_ref[...] = z - m - lse


def _tc_logsoftmax(q, b2, rows_per_core):
    body = functools.partial(_lsm_body, rows_a=rows_per_core,
                             rows_b=rows_per_core + NUM_TARGET)
    return pl.pallas_call(
        body,
        grid=(1,),
        in_specs=[
            pl.BlockSpec(q.shape, lambda i: (0, 0)),
            pl.BlockSpec((1, D_OUT), lambda i: (0, 0)),
        ],
        out_specs=pl.BlockSpec((NUM_TARGET, D_OUT), lambda i: (0, 0)),
        out_shape=jax.ShapeDtypeStruct((NUM_TARGET, D_OUT), jnp.float32),
    )(q, b2.reshape(1, D_OUT))


def _sc_agg_body(n_rows, acc_rows, d, ch_a, ch_b, out_tiles,
                 h_hbm, src_hbm, dst_hbm, zeros_hbm, out_hbm,
                 idx_s0, idx_s1, idx_d0, idx_d1, rows0, rows1,
                 acc, isem0, isem1, gsem0, gsem1):
    c = lax.axis_index("c")
    s = lax.axis_index("s")
    ch = jnp.where(c == 0, ch_a, ch_b)
    zrows = acc_rows // NS
    orows = n_rows // out_tiles
    pltpu.sync_copy(zeros_hbm, acc.at[pl.ds(s * zrows, zrows)])
    plsc.subcore_barrier()

    base = jnp.where(c == 0, s * ch_a, NS * ch_a + s * ch_b) * CHUNK
    idx_s = (idx_s0, idx_s1)
    idx_d = (idx_d0, idx_d1)
    rows = (rows0, rows1)
    isems = (isem0, isem1)
    gsems = (gsem0, gsem1)

    def fire_idx(k, b):
        off = base + k * CHUNK
        pltpu.async_copy(src_hbm.at[pl.ds(off, CHUNK)], idx_s[b], isems[b])
        pltpu.async_copy(dst_hbm.at[pl.ds(off, CHUNK)], idx_d[b], isems[b])

    def wait_idx(k, b):
        off = base + k * CHUNK
        pltpu.make_async_copy(src_hbm.at[pl.ds(off, CHUNK)], idx_s[b],
                              isems[b]).wait()
        pltpu.make_async_copy(dst_hbm.at[pl.ds(off, CHUNK)], idx_d[b],
                              isems[b]).wait()

    fire_idx(0, 0)
    wait_idx(0, 0)
    pltpu.async_copy(h_hbm.at[idx_s[0]], rows[0], gsems[0])
    fire_idx(1, 1)

    @pl.loop(0, ch, step=2)
    def _(k):
        for b in range(2):
            kk = k + b
            nb = 1 - b

            @pl.when(kk + 1 < ch)
            def _():
                wait_idx(kk + 1, nb)
                pltpu.async_copy(h_hbm.at[idx_s[nb]], rows[nb], gsems[nb])

            pltpu.make_async_copy(h_hbm.at[idx_s[b]], rows[b],
                                  gsems[b]).wait()
            pltpu.sync_copy(rows[b], acc.at[idx_d[b]], add=True)

            @pl.when(kk + 2 < ch)
            def _():
                fire_idx(kk + 2, b)

    plsc.subcore_barrier()

    @pl.when(s < out_tiles)
    def _():
        pltpu.sync_copy(acc.at[pl.ds(s * orows, orows)],
                        out_hbm.at[pl.ds((c * n_rows) + s * orows, orows)])


def _sc_aggregate(h, src, dst, n_rows, acc_rows, ch_a, ch_b, out_tiles):
    d = h.shape[1]
    zrows = acc_rows // NS
    zeros = jnp.zeros((zrows, d), jnp.float32)
    mesh = plsc.VectorSubcoreMesh(core_axis_name="c", subcore_axis_name="s")
    body = functools.partial(_sc_agg_body, n_rows, acc_rows, d,
                             ch_a, ch_b, out_tiles)
    return pl.kernel(
        body,
        out_type=jax.ShapeDtypeStruct((NC * n_rows, d), jnp.float32),
        mesh=mesh,
        scratch_types=[
            pltpu.VMEM((CHUNK,), jnp.int32),
            pltpu.VMEM((CHUNK,), jnp.int32),
            pltpu.VMEM((CHUNK,), jnp.int32),
            pltpu.VMEM((CHUNK,), jnp.int32),
            pltpu.VMEM((CHUNK, d), jnp.float32),
            pltpu.VMEM((CHUNK, d), jnp.float32),
            pltpu.VMEM_SHARED((acc_rows, d), jnp.float32),
            pltpu.SemaphoreType.DMA,
            pltpu.SemaphoreType.DMA,
            pltpu.SemaphoreType.DMA,
            pltpu.SemaphoreType.DMA,
        ],
    )(h, src, dst, zeros)


def _pad_edges(src, dst, e, e_pad, junk_lo, junk_n):
    pad = e_pad - e
    junk = junk_lo + (jnp.arange(pad, dtype=jnp.int32) % junk_n)
    srcp = jnp.concatenate([src, jnp.zeros((pad,), jnp.int32)])
    dstp = jnp.concatenate([dst, junk])
    return srcp, dstp


def kernel(x, edge_index_1, edge_index_2, num_target,
           W1, b1, gamma1, beta1, W2, b2):
    eps = 1e-5
    scale = gamma1 / jnp.sqrt(1.0 + eps)
    w1s = W1 * scale[None, :]
    cvec = (b1 * scale + beta1).reshape(1, D_HID)

    e1 = edge_index_1.shape[1]
    e2 = edge_index_2.shape[1]
    frac0 = 0.70

    def _split(e):
        total = _ceil_div(e, NS * CHUNK)
        a = max(2, int(round(total * frac0 / 2)) * 2)
        b = max(2, _ceil_div(total - a, 2) * 2)
        return a, b

    ch1a, ch1b = _split(e1)
    ch2a, ch2b = _split(e2)
    e1p = NS * (ch1a + ch1b) * CHUNK
    e2p = NS * (ch2a + ch2b) * CHUNK

    acc1_rows = _ceil_div(N + 1, NS * 8) * NS * 8
    src1, dst1 = _pad_edges(edge_index_1[0], edge_index_1[1], e1, e1p,
                            N, acc1_rows - N)

    acc2_rows = 2048
    d2 = edge_index_2[1]
    dst2 = jnp.where(d2 < NUM_TARGET, d2,
                     NUM_TARGET + lax.rem(d2, acc2_rows - NUM_TARGET))
    src2, dst2 = _pad_edges(edge_index_2[0], dst2, e2, e2p,
                            NUM_TARGET, acc2_rows - NUM_TARGET)

    w2p = jnp.pad(W2, ((0, 0), (0, 128 - D_OUT)))

    h = _tc_matmul(x, w1s, bm=1000)
    parts1 = _sc_aggregate(h, src1, dst1, N, acc1_rows, ch1a, ch1b,
                           10)
    h2 = _tc_relu_matmul(parts1, cvec, w2p, bm=1000)
    parts2 = _sc_aggregate(h2, src2, dst2, acc2_rows, acc2_rows,
                           ch2a, ch2b, NS)
    return _tc_logsoftmax(parts2, b2, acc2_rows)


# speedup vs baseline: 4.1501x; 1.0077x over previous
import functools

import jax
import jax.numpy as jnp
from jax import lax
from jax.experimental import pallas as pl
from jax.experimental.pallas import tpu as pltpu
from jax.experimental.pallas import tpu_sc as plsc

N = 10000
D_IN = 128
D_HID = 128
D_OUT = 40
NUM_TARGET = 1000

NC = 2
NS = 16
CHUNK = 128


def _ceil_div(a, b):
    return -(-a // b)


def _mm_body(x_ref, w_ref, o_ref):
    o_ref[...] = jnp.dot(x_ref[...], w_ref[...],
                         preferred_element_type=jnp.float32)


def _tc_matmul(x, w, bm):
    m, k = x.shape
    n = w.shape[1]
    grid = m // bm
    return pl.pallas_call(
        _mm_body,
        grid=(grid,),
        in_specs=[
            pl.BlockSpec((bm, k), lambda i: (i, 0)),
            pl.BlockSpec((k, n), lambda i: (0, 0)),
        ],
        out_specs=pl.BlockSpec((bm, n), lambda i: (i, 0)),
        out_shape=jax.ShapeDtypeStruct((m, n), jnp.float32),
    )(x, w)


def _relu_mm_body(p0_ref, p1_ref, c_ref, w_ref, o_ref):
    a = jnp.maximum(p0_ref[...] + p1_ref[...] + c_ref[...], 0.0)
    o_ref[...] = jnp.dot(a, w_ref[...], preferred_element_type=jnp.float32)


def _tc_relu_matmul(parts, cvec, w, bm):
    d = parts.shape[1]
    n = w.shape[1]
    grid = N // bm
    nb = N // bm
    return pl.pallas_call(
        _relu_mm_body,
        grid=(grid,),
        in_specs=[
            pl.BlockSpec((bm, d), lambda i: (i, 0)),
            pl.BlockSpec((bm, d), lambda i, nb=nb: (i + nb, 0)),
            pl.BlockSpec((1, d), lambda i: (0, 0)),
            pl.BlockSpec((d, n), lambda i: (0, 0)),
        ],
        out_specs=pl.BlockSpec((bm, n), lambda i: (i, 0)),
        out_shape=jax.ShapeDtypeStruct((N, n), jnp.float32),
    )(parts, parts, cvec, w)


def _lsm_body(q_ref, b_ref, o_ref, rows_a, rows_b):
    z = (q_ref[0:NUM_TARGET, 0:D_OUT] + q_ref[rows_a:rows_b, 0:D_OUT]
         + b_ref[...])
    m = jnp.max(z, axis=-1, keepdims=True)
    e = jnp.exp(z - m)
    lse = jnp.log(jnp.sum(e, axis=-1, keepdims=True))
    o_ref[...] = z - m - lse


def _tc_logsoftmax(q, b2, rows_per_core):
    body = functools.partial(_lsm_body, rows_a=rows_per_core,
                             rows_b=rows_per_core + NUM_TARGET)
    return pl.pallas_call(
        body,
        grid=(1,),
        in_specs=[
            pl.BlockSpec(q.shape, lambda i: (0, 0)),
            pl.BlockSpec((1, D_OUT), lambda i: (0, 0)),
        ],
        out_specs=pl.BlockSpec((NUM_TARGET, D_OUT), lambda i: (0, 0)),
        out_shape=jax.ShapeDtypeStruct((NUM_TARGET, D_OUT), jnp.float32),
    )(q, b2.reshape(1, D_OUT))


def _sc_agg_body(n_rows, acc_rows, d, ch_a, ch_b, out_tiles,
                 h_hbm, src_hbm, dst_hbm, zeros_hbm, out_hbm,
                 idx_s0, idx_s1, idx_d0, idx_d1, rows0, rows1,
                 acc, isem0, isem1, gsem0, gsem1):
    c = lax.axis_index("c")
    s = lax.axis_index("s")
    ch = jnp.where(c == 0, ch_a, ch_b)
    zrows = acc_rows // NS
    orows = n_rows // out_tiles
    pltpu.sync_copy(zeros_hbm, acc.at[pl.ds(s * zrows, zrows)])
    plsc.subcore_barrier()

    base = jnp.where(c == 0, s * ch_a, NS * ch_a + s * ch_b) * CHUNK
    idx_s = (idx_s0, idx_s1)
    idx_d = (idx_d0, idx_d1)
    rows = (rows0, rows1)
    isems = (isem0, isem1)
    gsems = (gsem0, gsem1)

    def fire_idx(k, b):
        off = base + k * CHUNK
        pltpu.async_copy(src_hbm.at[pl.ds(off, CHUNK)], idx_s[b], isems[b])
        pltpu.async_copy(dst_hbm.at[pl.ds(off, CHUNK)], idx_d[b], isems[b])

    def wait_idx(k, b):
        off = base + k * CHUNK
        pltpu.make_async_copy(src_hbm.at[pl.ds(off, CHUNK)], idx_s[b],
                              isems[b]).wait()
        pltpu.make_async_copy(dst_hbm.at[pl.ds(off, CHUNK)], idx_d[b],
                              isems[b]).wait()

    fire_idx(0, 0)
    wait_idx(0, 0)
    pltpu.async_copy(h_hbm.at[idx_s[0]], rows[0], gsems[0])
    fire_idx(1, 1)

    @pl.loop(0, ch, step=2)
    def _(k):
        for b in range(2):
            kk = k + b
            nb = 1 - b

            @pl.when(kk + 1 < ch)
            def _():
                wait_idx(kk + 1, nb)
                pltpu.async_copy(h_hbm.at[idx_s[nb]], rows[nb], gsems[nb])

            pltpu.make_async_copy(h_hbm.at[idx_s[b]], rows[b],
                                  gsems[b]).wait()
            pltpu.sync_copy(rows[b], acc.at[idx_d[b]], add=True)

            @pl.when(kk + 2 < ch)
            def _():
                fire_idx(kk + 2, b)

    plsc.subcore_barrier()

    @pl.when(s < out_tiles)
    def _():
        pltpu.sync_copy(acc.at[pl.ds(s * orows, orows)],
                        out_hbm.at[pl.ds((c * n_rows) + s * orows, orows)])


def _sc_aggregate(h, src, dst, n_rows, acc_rows, ch_a, ch_b, out_tiles):
    d = h.shape[1]
    zrows = acc_rows // NS
    zeros = jnp.zeros((zrows, d), jnp.float32)
    mesh = plsc.VectorSubcoreMesh(core_axis_name="c", subcore_axis_name="s")
    body = functools.partial(_sc_agg_body, n_rows, acc_rows, d,
                             ch_a, ch_b, out_tiles)
    return pl.kernel(
        body,
        out_type=jax.ShapeDtypeStruct((NC * n_rows, d), jnp.float32),
        mesh=mesh,
        scratch_types=[
            pltpu.VMEM((CHUNK,), jnp.int32),
            pltpu.VMEM((CHUNK,), jnp.int32),
            pltpu.VMEM((CHUNK,), jnp.int32),
            pltpu.VMEM((CHUNK,), jnp.int32),
            pltpu.VMEM((CHUNK, d), jnp.float32),
            pltpu.VMEM((CHUNK, d), jnp.float32),
            pltpu.VMEM_SHARED((acc_rows, d), jnp.float32),
            pltpu.SemaphoreType.DMA,
            pltpu.SemaphoreType.DMA,
            pltpu.SemaphoreType.DMA,
            pltpu.SemaphoreType.DMA,
        ],
    )(h, src, dst, zeros)


def _pad_edges(src, dst, e, e_pad, junk_lo, junk_n):
    pad = e_pad - e
    junk = junk_lo + (jnp.arange(pad, dtype=jnp.int32) % junk_n)
    srcp = jnp.concatenate([src, jnp.zeros((pad,), jnp.int32)])
    dstp = jnp.concatenate([dst, junk])
    return srcp, dstp


def kernel(x, edge_index_1, edge_index_2, num_target,
           W1, b1, gamma1, beta1, W2, b2):
    eps = 1e-5
    scale = gamma1 / jnp.sqrt(1.0 + eps)
    w1s = W1 * scale[None, :]
    cvec = (b1 * scale + beta1).reshape(1, D_HID)

    e1 = edge_index_1.shape[1]
    e2 = edge_index_2.shape[1]
    def _split(e, frac0):
        total = _ceil_div(e, NS * CHUNK)
        a = max(2, int(round(total * frac0 / 2)) * 2)
        b = max(2, _ceil_div(total - a, 2) * 2)
        return a, b

    ch1a, ch1b = _split(e1, 0.74)
    ch2a, ch2b = _split(e2, 0.82)
    e1p = NS * (ch1a + ch1b) * CHUNK
    e2p = NS * (ch2a + ch2b) * CHUNK

    acc1_rows = _ceil_div(N + 1, NS * 8) * NS * 8
    src1, dst1 = _pad_edges(edge_index_1[0], edge_index_1[1], e1, e1p,
                            N, acc1_rows - N)

    acc2_rows = 2048
    d2 = edge_index_2[1]
    dst2 = jnp.where(d2 < NUM_TARGET, d2,
                     NUM_TARGET + lax.rem(d2, acc2_rows - NUM_TARGET))
    src2, dst2 = _pad_edges(edge_index_2[0], dst2, e2, e2p,
                            NUM_TARGET, acc2_rows - NUM_TARGET)

    w2p = jnp.pad(W2, ((0, 0), (0, 128 - D_OUT)))

    h = _tc_matmul(x, w1s, bm=1000)
    parts1 = _sc_aggregate(h, src1, dst1, N, acc1_rows, ch1a, ch1b,
                           10)
    h2 = _tc_relu_matmul(parts1, cvec, w2p, bm=1000)
    parts2 = _sc_aggregate(h2, src2, dst2, acc2_rows, acc2_rows,
                           ch2a, ch2b, NS)
    return _tc_logsoftmax(parts2, b2, acc2_rows)
